# Initial kernel scaffold; baseline (speedup 1.0000x reference)
import jax
import jax.numpy as jnp
from jax.experimental import pallas as pl


def kernel(codes, codebook, W, b, ln_gamma, ln_beta):
    raise NotImplementedError("write your pallas kernel here")


# speedup vs baseline: 6.1711x; 6.1711x over previous
import functools

import jax
import jax.numpy as jnp
from jax import lax
from jax.experimental import pallas as pl
from jax.experimental.pallas import tpu as pltpu
from jax.experimental.pallas import tpu_sc as plsc

LN_EPS = 1e-5
LANES = 16
NC = 2
NS = 16
NW = NC * NS
KCODES = 8
D = 128
CHUNK = 32


def _proj_body(cb_ref, w_ref, b_ref, o_ref):
    o_ref[...] = (
        jnp.dot(cb_ref[...], w_ref[...], preferred_element_type=jnp.float32)
        + b_ref[...]
    ) * 0.125


def _ln_body(x_ref, g_ref, bb_ref, o_ref):
    x = x_ref[...]
    mu = jnp.mean(x, axis=-1, keepdims=True)
    xc = x - mu
    var = jnp.mean(xc * xc, axis=-1, keepdims=True)
    o_ref[...] = xc * lax.rsqrt(var + LN_EPS) * g_ref[...] + bb_ref[...]


OUTER = 128
NSUB = OUTER // CHUNK


@functools.lru_cache(maxsize=None)
def _make_sc_gather(T):
    TPW = T // NW
    NOUT = TPW // OUTER
    mesh = plsc.VectorSubcoreMesh(core_axis_name="c", subcore_axis_name="s")

    @functools.partial(
        pl.kernel,
        mesh=mesh,
        out_type=jax.ShapeDtypeStruct((T, D), jnp.float32),
        scratch_types=[
            pltpu.VMEM((KCODES, 128), jnp.int32),
            pltpu.VMEM((CHUNK * KCODES, D), jnp.float32),
            pltpu.VMEM((OUTER, D), jnp.float32),
            pltpu.SemaphoreType.DMA,
        ],
    )
    def sc_gather(p_hbm, codes_hbm, out_hbm, idx_v, rows_v, acc_v, sem):
        wid = lax.axis_index("s") * NC + lax.axis_index("c")

        def chunk_body(j, carry):
            base_tok = pl.multiple_of(wid * TPW + j * OUTER, OUTER)
            idx_row = pl.multiple_of(base_tok // (128 // KCODES), KCODES)
            pltpu.sync_copy(codes_hbm.at[pl.ds(idx_row, KCODES)], idx_v)
            for sub in range(NSUB):
                cp0 = pltpu.async_copy(
                    p_hbm.at[idx_v.at[2 * sub]], rows_v.at[pl.ds(0, 128)], sem)
                cp1 = pltpu.async_copy(
                    p_hbm.at[idx_v.at[2 * sub + 1]],
                    rows_v.at[pl.ds(128, 128)], sem)
                cp0.wait()
                cp1.wait()

                def tok_body(i, carry2):
                    r = i * KCODES
                    for jj in range(D // LANES):
                        sl = pl.ds(jj * LANES, LANES)
                        a = rows_v[r, sl]
                        for k in range(1, KCODES):
                            a = a + rows_v[r + k, sl]
                        acc_v[sub * CHUNK + i, sl] = a
                    return carry2

                lax.fori_loop(0, CHUNK, tok_body, 0)
            pltpu.sync_copy(acc_v, out_hbm.at[pl.ds(base_tok, OUTER)])
            return carry

        lax.fori_loop(0, NOUT, chunk_body, 0)

    return sc_gather


def kernel(codes, codebook, W, b, ln_gamma, ln_beta):
    Bb, Nn, Kk = codes.shape
    T = Bb * Nn
    codes_flat = codes.astype(jnp.int32).reshape(T * Kk // 128, 128)
    cb_pad = jnp.pad(codebook, ((0, 7), (0, 0)))
    P = pl.pallas_call(
        _proj_body,
        out_shape=jax.ShapeDtypeStruct((cb_pad.shape[0], D), jnp.float32),
    )(cb_pad, W, b.reshape(1, D))
    sums = _make_sc_gather(T)(P, codes_flat)
    TB = 8192
    out = pl.pallas_call(
        _ln_body,
        grid=(T // TB,),
        in_specs=[
            pl.BlockSpec((TB, D), lambda i: (i, 0)),
            pl.BlockSpec((1, D), lambda i: (0, 0)),
            pl.BlockSpec((1, D), lambda i: (0, 0)),
        ],
        out_specs=pl.BlockSpec((TB, D), lambda i: (i, 0)),
        out_shape=jax.ShapeDtypeStruct((T, D), jnp.float32),
    )(sums, ln_gamma.reshape(1, D), ln_beta.reshape(1, D))
    return out.reshape(Bb, Nn, D)


# speedup vs baseline: 7.3407x; 1.1895x over previous
import functools

import jax
import jax.numpy as jnp
from jax import lax
from jax.experimental import pallas as pl
from jax.experimental.pallas import tpu as pltpu
from jax.experimental.pallas import tpu_sc as plsc

LN_EPS = 1e-5
LANES = 16
NC = 2
NS = 16
NW = NC * NS
KCODES = 8
D = 128
CHUNK = 32


def _proj_body(cb_ref, w_ref, b_ref, o_ref):
    o_ref[...] = (
        jnp.dot(cb_ref[...], w_ref[...], preferred_element_type=jnp.float32)
        + b_ref[...]
    ) * 0.125


def _ln_body(x_ref, g_ref, bb_ref, o_ref):
    x = x_ref[...]
    mu = jnp.mean(x, axis=-1, keepdims=True)
    xc = x - mu
    var = jnp.mean(xc * xc, axis=-1, keepdims=True)
    o_ref[...] = xc * lax.rsqrt(var + LN_EPS) * g_ref[...] + bb_ref[...]


OUTER = 128
NSUB = OUTER // CHUNK


@functools.lru_cache(maxsize=None)
def _make_sc_gather(T):
    TPW = T // NW
    NOUT = TPW // OUTER
    mesh = plsc.VectorSubcoreMesh(core_axis_name="c", subcore_axis_name="s")

    @functools.partial(
        pl.kernel,
        mesh=mesh,
        out_type=jax.ShapeDtypeStruct((T, D), jnp.float32),
        scratch_types=[
            pltpu.VMEM((KCODES, 128), jnp.int32),
            pltpu.VMEM((3, CHUNK * KCODES, D), jnp.float32),
            pltpu.VMEM((OUTER, D), jnp.float32),
            pltpu.SemaphoreType.DMA,
            pltpu.SemaphoreType.DMA,
            pltpu.SemaphoreType.DMA,
        ],
    )
    def sc_gather(p_hbm, codes_hbm, out_hbm, idx_v, rows_v, acc_v,
                  sem0, sem1, sem2):
        wid = lax.axis_index("s") * NC + lax.axis_index("c")
        sems = (sem0, sem1, sem2)

        def issue(sub, buf):
            cp0 = pltpu.async_copy(
                p_hbm.at[idx_v.at[2 * sub]],
                rows_v.at[buf, pl.ds(0, 128)], sems[buf])
            cp1 = pltpu.async_copy(
                p_hbm.at[idx_v.at[2 * sub + 1]],
                rows_v.at[buf, pl.ds(128, 128)], sems[buf])
            return cp0, cp1

        def chunk_body(j, carry):
            base_tok = pl.multiple_of(wid * TPW + j * OUTER, OUTER)
            idx_row = pl.multiple_of(base_tok // (128 // KCODES), KCODES)
            pltpu.sync_copy(codes_hbm.at[pl.ds(idx_row, KCODES)], idx_v)
            pend = {0: issue(0, 0), 1: issue(1, 1), 2: issue(2, 2)}
            for sub in range(NSUB):
                buf = sub % 3
                cp0, cp1 = pend[sub]
                cp0.wait()
                cp1.wait()

                def tok_body(i, carry2, _sub=sub, _buf=buf):
                    r = i * KCODES
                    for jj in range(D // LANES):
                        sl = pl.ds(jj * LANES, LANES)
                        a = rows_v[_buf, r, sl]
                        for k in range(1, KCODES):
                            a = a + rows_v[_buf, r + k, sl]
                        acc_v[_sub * CHUNK + i, sl] = a
                    return carry2

                lax.fori_loop(0, CHUNK, tok_body, 0)
                if sub == 0:
                    pend[3] = issue(3, 0)
            pltpu.sync_copy(acc_v, out_hbm.at[pl.ds(base_tok, OUTER)])
            return carry

        lax.fori_loop(0, NOUT, chunk_body, 0)

    return sc_gather


def kernel(codes, codebook, W, b, ln_gamma, ln_beta):
    Bb, Nn, Kk = codes.shape
    T = Bb * Nn
    codes_flat = codes.astype(jnp.int32).reshape(T * Kk // 128, 128)
    cb_pad = jnp.pad(codebook, ((0, 7), (0, 0)))
    P = pl.pallas_call(
        _proj_body,
        out_shape=jax.ShapeDtypeStruct((cb_pad.shape[0], D), jnp.float32),
    )(cb_pad, W, b.reshape(1, D))
    sums = _make_sc_gather(T)(P, codes_flat)
    TB = 8192
    out = pl.pallas_call(
        _ln_body,
        grid=(T // TB,),
        in_specs=[
            pl.BlockSpec((TB, D), lambda i: (i, 0)),
            pl.BlockSpec((1, D), lambda i: (0, 0)),
            pl.BlockSpec((1, D), lambda i: (0, 0)),
        ],
        out_specs=pl.BlockSpec((TB, D), lambda i: (i, 0)),
        out_shape=jax.ShapeDtypeStruct((T, D), jnp.float32),
    )(sums, ln_gamma.reshape(1, D), ln_beta.reshape(1, D))
    return out.reshape(Bb, Nn, D)


# speedup vs baseline: 8.0313x; 1.0941x over previous
import functools

import jax
import jax.numpy as jnp
from jax import lax
from jax.experimental import pallas as pl
from jax.experimental.pallas import tpu as pltpu
from jax.experimental.pallas import tpu_sc as plsc

LN_EPS = 1e-5
LANES = 16
NC = 2
NS = 16
NW = NC * NS
KCODES = 8
D = 128
CHUNK = 32
OUTER = 128
NSUB = OUTER // CHUNK


def _proj_body(cb_ref, w_ref, b_ref, o_ref):
    o_ref[...] = (
        jnp.dot(cb_ref[...], w_ref[...], preferred_element_type=jnp.float32)
        + b_ref[...]
    ) * 0.125


def _ln_body(x_ref, g_ref, bb_ref, o_ref):
    x = x_ref[...]
    mu = jnp.mean(x, axis=-1, keepdims=True)
    xc = x - mu
    var = jnp.mean(xc * xc, axis=-1, keepdims=True)
    o_ref[...] = xc * lax.rsqrt(var + LN_EPS) * g_ref[...] + bb_ref[...]


@functools.lru_cache(maxsize=None)
def _make_sc_gather(T):
    TPW = T // NW
    NOUT = TPW // OUTER
    mesh = plsc.VectorSubcoreMesh(core_axis_name="c", subcore_axis_name="s")

    @functools.partial(
        pl.kernel,
        mesh=mesh,
        out_type=jax.ShapeDtypeStruct((T, D), jnp.float32),
        scratch_types=[
            pltpu.VMEM((KCODES, 128), jnp.int32),
            pltpu.VMEM((3, CHUNK * KCODES, D), jnp.float32),
            pltpu.VMEM((OUTER, D), jnp.float32),
            pltpu.SemaphoreType.DMA,
            pltpu.SemaphoreType.DMA,
            pltpu.SemaphoreType.DMA,
            pltpu.SemaphoreType.DMA,
        ],
    )
    def sc_gather(p_hbm, codes_hbm, out_hbm, idx_v, rows_v, acc_v,
                  sem0, sem1, sem2, out_sem):
        wid = lax.axis_index("s") * NC + lax.axis_index("c")
        sems = (sem0, sem1, sem2)

        def issue(sub, buf):
            cp0 = pltpu.async_copy(
                p_hbm.at[idx_v.at[2 * sub]],
                rows_v.at[buf, pl.ds(0, 128)], sems[buf])
            cp1 = pltpu.async_copy(
                p_hbm.at[idx_v.at[2 * sub + 1]],
                rows_v.at[buf, pl.ds(128, 128)], sems[buf])
            return cp0, cp1

        def out_copy(base_tok):
            return pltpu.make_async_copy(
                acc_v, out_hbm.at[pl.ds(base_tok, OUTER)], out_sem)

        def chunk_body(j, carry):
            base_tok = pl.multiple_of(wid * TPW + j * OUTER, OUTER)
            idx_row = pl.multiple_of(base_tok // (128 // KCODES), KCODES)
            pltpu.sync_copy(codes_hbm.at[pl.ds(idx_row, KCODES)], idx_v)
            pend = {0: issue(0, 0), 1: issue(1, 1), 2: issue(2, 2)}

            @pl.when(j > 0)
            def _():
                out_copy(base_tok - OUTER).wait()

            for sub in range(NSUB):
                buf = sub % 3
                cp0, cp1 = pend[sub]
                cp0.wait()
                cp1.wait()

                def pair_body(p, carry2, _sub=sub, _buf=buf):
                    for t in range(2):
                        i = p * 2 + t
                        r = i * KCODES
                        row = _sub * CHUNK + i
                        for jj in range(D // LANES):
                            sl = pl.ds(jj * LANES, LANES)
                            v = [rows_v[_buf, r + k, sl]
                                 for k in range(KCODES)]
                            s = (((v[0] + v[1]) + (v[2] + v[3]))
                                 + ((v[4] + v[5]) + (v[6] + v[7])))
                            acc_v[row, sl] = s
                    return carry2

                lax.fori_loop(0, CHUNK // 2, pair_body, 0)
                if sub == 0:
                    pend[3] = issue(3, 0)
            out_copy(base_tok).start()
            return carry

        lax.fori_loop(0, NOUT, chunk_body, 0)
        out_copy(wid * TPW + (NOUT - 1) * OUTER).wait()

    return sc_gather


def kernel(codes, codebook, W, b, ln_gamma, ln_beta):
    Bb, Nn, Kk = codes.shape
    T = Bb * Nn
    codes_flat = codes.astype(jnp.int32).reshape(T * Kk // 128, 128)
    cb_pad = jnp.pad(codebook, ((0, 7), (0, 0)))
    P = pl.pallas_call(
        _proj_body,
        out_shape=jax.ShapeDtypeStruct((cb_pad.shape[0], D), jnp.float32),
    )(cb_pad, W, b.reshape(1, D))
    sums = _make_sc_gather(T)(P, codes_flat)
    TB = 8192
    out = pl.pallas_call(
        _ln_body,
        grid=(T // TB,),
        in_specs=[
            pl.BlockSpec((TB, D), lambda i: (i, 0)),
            pl.BlockSpec((1, D), lambda i: (0, 0)),
            pl.BlockSpec((1, D), lambda i: (0, 0)),
        ],
        out_specs=pl.BlockSpec((TB, D), lambda i: (i, 0)),
        out_shape=jax.ShapeDtypeStruct((T, D), jnp.float32),
    )(sums, ln_gamma.reshape(1, D), ln_beta.reshape(1, D))
    return out.reshape(Bb, Nn, D)


# speedup vs baseline: 10.3108x; 1.2838x over previous
import functools

import jax
import jax.numpy as jnp
from jax import lax
from jax.experimental import pallas as pl
from jax.experimental.pallas import tpu as pltpu
from jax.experimental.pallas import tpu_sc as plsc

LN_EPS = 1e-5
LANES = 16
NC = 2
NS = 16
NW = NC * NS
KCODES = 8
D = 128
SUB = 16
OUTER = 128
NSUB = OUTER // SUB


def _proj_body(cb_ref, w_ref, b_ref, o_ref):
    o_ref[...] = (
        jnp.dot(cb_ref[...], w_ref[...], preferred_element_type=jnp.float32)
        + b_ref[...]
    ) * 0.125


def _ln_body(x_ref, g_ref, bb_ref, o_ref):
    x = x_ref[...]
    mu = jnp.mean(x, axis=-1, keepdims=True)
    xc = x - mu
    var = jnp.mean(xc * xc, axis=-1, keepdims=True)
    o_ref[...] = xc * lax.rsqrt(var + LN_EPS) * g_ref[...] + bb_ref[...]


@functools.lru_cache(maxsize=None)
def _make_sc_gather(T):
    TPW = T // NW
    NOUT = TPW // OUTER
    mesh = plsc.VectorSubcoreMesh(core_axis_name="c", subcore_axis_name="s")

    @functools.partial(
        pl.kernel,
        mesh=mesh,
        out_type=jax.ShapeDtypeStruct((T, D), jnp.float32),
        scratch_types=[
            pltpu.VMEM((2, KCODES, 128), jnp.int32),
            pltpu.VMEM((4, SUB * KCODES, D), jnp.float32),
            pltpu.VMEM((OUTER, D), jnp.float32),
            pltpu.SemaphoreType.DMA,
            pltpu.SemaphoreType.DMA,
            pltpu.SemaphoreType.DMA,
            pltpu.SemaphoreType.DMA,
            pltpu.SemaphoreType.DMA,
            pltpu.SemaphoreType.DMA,
        ],
    )
    def sc_gather(p_hbm, codes_hbm, out_hbm, idx_v, rows_v, acc_v,
                  sem0, sem1, sem2, sem3, idx_sem, out_sem):
        wid = lax.axis_index("s") * NC + lax.axis_index("c")
        sems = (sem0, sem1, sem2, sem3)
        base0 = pl.multiple_of(wid * TPW, OUTER)

        def idx_row_of(jo):
            return pl.multiple_of((base0 + jo * OUTER) // (128 // KCODES),
                                  KCODES)

        def issue(jb, sub, buf):
            pltpu.async_copy(
                p_hbm.at[idx_v.at[jb, sub]], rows_v.at[buf], sems[buf])

        def wait_rows(buf):
            pltpu.make_async_copy(
                p_hbm.at[idx_v.at[0, 0]], rows_v.at[buf], sems[buf]).wait()

        def idx_prefetch(jo, jb):
            return pltpu.make_async_copy(
                codes_hbm.at[pl.ds(idx_row_of(jo), KCODES)],
                idx_v.at[jb], idx_sem)

        def out_copy(base_tok):
            return pltpu.make_async_copy(
                acc_v, out_hbm.at[pl.ds(base_tok, OUTER)], out_sem)

        pltpu.sync_copy(codes_hbm.at[pl.ds(idx_row_of(0), KCODES)],
                        idx_v.at[0])
        for s in range(4):
            issue(0, s, s)
        if NOUT > 1:
            idx_prefetch(1, 1).start()

        def chunk_body(j, carry):
            base_tok = pl.multiple_of(base0 + j * OUTER, OUTER)
            jb = lax.rem(j, 2)
            jb_next = lax.rem(j + 1, 2)

            @pl.when(j > 0)
            def _():
                out_copy(base_tok - OUTER).wait()

            for sub in range(NSUB):
                buf = sub % 4
                wait_rows(buf)

                def pair_body(p, carry2, _sub=sub, _buf=buf):
                    for t in range(2):
                        i = p * 2 + t
                        r = i * KCODES
                        row = _sub * SUB + i
                        for jj in range(D // LANES):
                            sl = pl.ds(jj * LANES, LANES)
                            v = [rows_v[_buf, r + k, sl]
                                 for k in range(KCODES)]
                            s = (((v[0] + v[1]) + (v[2] + v[3]))
                                 + ((v[4] + v[5]) + (v[6] + v[7])))
                            acc_v[row, sl] = s
                    return carry2

                lax.fori_loop(0, SUB // 2, pair_body, 0)

                if sub < 4:
                    issue(jb, sub + 4, buf)
                else:
                    if sub == 4:
                        @pl.when(j + 1 < NOUT)
                        def _():
                            idx_prefetch(0, 0).wait()
                            issue(jb_next, 0, buf)
                    else:
                        @pl.when(j + 1 < NOUT)
                        def _():
                            issue(jb_next, sub - 4, buf)
                    if sub == 7:
                        @pl.when(j + 2 < NOUT)
                        def _():
                            idx_prefetch(j + 2, jb).start()
            out_copy(base_tok).start()
            return carry

        lax.fori_loop(0, NOUT, chunk_body, 0)
        out_copy(base0 + (NOUT - 1) * OUTER).wait()

    return sc_gather


def kernel(codes, codebook, W, b, ln_gamma, ln_beta):
    Bb, Nn, Kk = codes.shape
    T = Bb * Nn
    codes_flat = codes.astype(jnp.int32).reshape(T * Kk // 128, 128)
    cb_pad = jnp.pad(codebook, ((0, 7), (0, 0)))
    P = pl.pallas_call(
        _proj_body,
        out_shape=jax.ShapeDtypeStruct((cb_pad.shape[0], D), jnp.float32),
    )(cb_pad, W, b.reshape(1, D))
    sums = _make_sc_gather(T)(P, codes_flat)
    TB = 8192
    out = pl.pallas_call(
        _ln_body,
        grid=(T // TB,),
        in_specs=[
            pl.BlockSpec((TB, D), lambda i: (i, 0)),
            pl.BlockSpec((1, D), lambda i: (0, 0)),
            pl.BlockSpec((1, D), lambda i: (0, 0)),
        ],
        out_specs=pl.BlockSpec((TB, D), lambda i: (i, 0)),
        out_shape=jax.ShapeDtypeStruct((T, D), jnp.float32),
    )(sums, ln_gamma.reshape(1, D), ln_beta.reshape(1, D))
    return out.reshape(Bb, Nn, D)
